# Initial kernel scaffold; baseline (speedup 1.0000x reference)
import jax
import jax.numpy as jnp
from jax.experimental import pallas as pl


def kernel(x, edge_index, W1, b1, W2, b2, W3, b3):
    raise NotImplementedError("write your pallas kernel here")


# speedup vs baseline: 5.9686x; 5.9686x over previous
import functools
import jax
import jax.numpy as jnp
from jax import lax
from jax.experimental import pallas as pl
from jax.experimental.pallas import tpu as pltpu
from jax.experimental.pallas import tpu_sc as plsc

N = 10000
E = 320000
D = 128

NPAD = 10240
EP = 327680
NCORES = 2
NSUB = 16
NTILES = NCORES * NSUB
EPT = EP // NTILES
CH = 128
NCH = EPT // CH
ROWS_PER_TILE = NPAD // NSUB


def _zero_vmem_2d(ref, nrows, ncols, val):
    v = jnp.full((16,), val, jnp.float32)

    @pl.loop(0, nrows)
    def _(r):
        @pl.loop(0, ncols // 16)
        def _(c):
            ref[r, pl.ds(c * 16, 16)] = v


def _copy_acc_out(acc_sh, out_hbm, cid, sid, rows_v):
    base = sid * ROWS_PER_TILE
    for k in range(5):
        pltpu.sync_copy(acc_sh.at[pl.ds(base + k * CH, CH)], rows_v)
        pltpu.sync_copy(rows_v, out_hbm.at[cid, pl.ds(base + k * CH, CH)])


def _make_agg_kernel(ncols, gather_table):
    mesh = plsc.VectorSubcoreMesh(core_axis_name="c", subcore_axis_name="s")

    def body(*refs):
        if gather_table:
            table_hbm, src_hbm, dst_hbm, out_hbm, idx_v, rows_v, acc_sh, sem = refs
        else:
            dst_hbm, out_hbm, idx_v, rows_v, acc_sh, sem = refs
        cid = lax.axis_index("c")
        sid = lax.axis_index("s")

        _zero_vmem_2d(rows_v, CH, ncols, 0.0)
        zbase = sid * ROWS_PER_TILE
        for k in range(5):
            pltpu.sync_copy(rows_v, acc_sh.at[pl.ds(zbase + k * CH, CH)])
        if not gather_table:
            _zero_vmem_2d(rows_v, CH, ncols, 1.0)
        plsc.subcore_barrier()

        ebase = (cid * NSUB + sid) * EPT

        @pl.loop(0, NCH)
        def _(ch):
            off = ebase + ch * CH
            if gather_table:
                pltpu.sync_copy(src_hbm.at[pl.ds(off, CH)], idx_v.at[0])
                pltpu.sync_copy(dst_hbm.at[pl.ds(off, CH)], idx_v.at[1])
                pltpu.async_copy(table_hbm.at[idx_v.at[0]], rows_v, sem).wait()
            else:
                pltpu.sync_copy(dst_hbm.at[pl.ds(off, CH)], idx_v.at[1])
            pltpu.sync_copy(rows_v, acc_sh.at[idx_v.at[1]], add=True)

        plsc.subcore_barrier()
        _copy_acc_out(acc_sh, out_hbm, cid, sid, rows_v)

    return pl.kernel(
        body,
        out_type=jax.ShapeDtypeStruct((NCORES, NPAD, ncols), jnp.float32),
        mesh=mesh,
        scratch_types=[
            pltpu.VMEM((2, CH), jnp.int32),
            pltpu.VMEM((CH, ncols), jnp.float32),
            pltpu.VMEM_SHARED((NPAD, ncols), jnp.float32),
            pltpu.SemaphoreType.DMA,
        ],
    )


RB = 2560
GRID = NPAD // RB


def _dinv_scale_body(dp0_ref, dp1_ref, h_ref, dinv_ref, t_ref):
    i = pl.program_id(0)
    deg = dp0_ref[:, :1] + dp1_ref[:, :1] + 1.0
    row = lax.broadcasted_iota(jnp.int32, (RB, 1), 0) + i * RB
    dinv = jnp.where(row < N, lax.rsqrt(deg), 0.0)
    dinv_ref[...] = dinv
    t_ref[...] = h_ref[...] * dinv


def _matmul_body(x_ref, w_ref, o_ref):
    o_ref[...] = jnp.dot(x_ref[...], w_ref[...], preferred_element_type=jnp.float32)


def _combine_body(a0_ref, a1_ref, t_ref, dinv_ref, b_ref, w_ref, o_ref, *, relu_next):
    dinv = dinv_ref[...]
    u = (a0_ref[0] + a1_ref[0] + t_ref[...]) * dinv + b_ref[...]
    if relu_next:
        u = jnp.maximum(u, 0.0)
        o_ref[...] = jnp.dot(u, w_ref[...], preferred_element_type=jnp.float32) * dinv
    else:
        o_ref[...] = u


def _row_spec():
    return pl.BlockSpec((RB, D), lambda i: (i, 0))


def _tc_matmul(x, w):
    return pl.pallas_call(
        _matmul_body,
        grid=(GRID,),
        in_specs=[_row_spec(), pl.BlockSpec((D, D), lambda i: (0, 0))],
        out_specs=_row_spec(),
        out_shape=jax.ShapeDtypeStruct((NPAD, D), jnp.float32),
    )(x, w)


def _tc_dinv_scale(dp0, dp1, h):
    return pl.pallas_call(
        _dinv_scale_body,
        grid=(GRID,),
        in_specs=[
            pl.BlockSpec((RB, 16), lambda i: (i, 0)),
            pl.BlockSpec((RB, 16), lambda i: (i, 0)),
            _row_spec(),
        ],
        out_specs=[pl.BlockSpec((RB, 1), lambda i: (i, 0)), _row_spec()],
        out_shape=[
            jax.ShapeDtypeStruct((NPAD, 1), jnp.float32),
            jax.ShapeDtypeStruct((NPAD, D), jnp.float32),
        ],
    )(dp0, dp1, h)


def _tc_combine(a, t, dinv, b2d, w, relu_next):
    return pl.pallas_call(
        functools.partial(_combine_body, relu_next=relu_next),
        grid=(GRID,),
        in_specs=[
            pl.BlockSpec((1, RB, D), lambda i: (0, i, 0)),
            pl.BlockSpec((1, RB, D), lambda i: (1, i, 0)),
            _row_spec(),
            pl.BlockSpec((RB, 1), lambda i: (i, 0)),
            pl.BlockSpec((1, D), lambda i: (0, 0)),
            pl.BlockSpec((D, D), lambda i: (0, 0)),
        ],
        out_specs=_row_spec(),
        out_shape=jax.ShapeDtypeStruct((NPAD, D), jnp.float32),
    )(a, a, t, dinv, b2d, w)


_agg_rows = _make_agg_kernel(D, gather_table=True)
_agg_deg = _make_agg_kernel(16, gather_table=False)


@jax.jit
def kernel(x, edge_index, W1, b1, W2, b2, W3, b3):
    src = edge_index[0].astype(jnp.int32)
    dst = edge_index[1].astype(jnp.int32)
    padi = jnp.full((EP - E,), N, jnp.int32)
    src_p = jnp.concatenate([src, padi])
    dst_p = jnp.concatenate([dst, padi])

    x_p = jnp.zeros((NPAD, D), jnp.float32).at[:N].set(x)
    b1r = b1.reshape(1, D)
    b2r = b2.reshape(1, D)
    b3r = b3.reshape(1, D)

    degp = _agg_deg(dst_p)
    h1 = _tc_matmul(x_p, W1)
    dinv, t1 = _tc_dinv_scale(degp[0], degp[1], h1)

    a1 = _agg_rows(t1, src_p, dst_p)
    t2 = _tc_combine(a1, t1, dinv, b1r, W2, relu_next=True)

    a2 = _agg_rows(t2, src_p, dst_p)
    t3 = _tc_combine(a2, t2, dinv, b2r, W3, relu_next=True)

    a3 = _agg_rows(t3, src_p, dst_p)
    out = _tc_combine(a3, t3, dinv, b3r, W3, relu_next=False)
    return out[:N]


# speedup vs baseline: 13.7902x; 2.3105x over previous
import functools
import jax
import jax.numpy as jnp
from jax import lax
from jax.experimental import pallas as pl
from jax.experimental.pallas import tpu as pltpu
from jax.experimental.pallas import tpu_sc as plsc

N = 10000
E = 320000
D = 128

NPAD = 10240
EP = 327680
NCORES = 2
NSUB = 16
NTILES = NCORES * NSUB
EPT = EP // NTILES
CH = 128
NCH = EPT // CH
ROWS_PER_TILE = NPAD // NSUB


def _zero_vmem_2d(ref, nrows, ncols, val):
    v = jnp.full((16,), val, jnp.float32)

    @pl.loop(0, nrows)
    def _(r):
        @pl.loop(0, ncols // 16)
        def _(c):
            ref[r, pl.ds(c * 16, 16)] = v


def _copy_acc_out(acc_sh, out_hbm, cid, sid, rows_v):
    base = sid * ROWS_PER_TILE
    for k in range(5):
        pltpu.sync_copy(acc_sh.at[pl.ds(base + k * CH, CH)], rows_v)
        pltpu.sync_copy(rows_v, out_hbm.at[cid, pl.ds(base + k * CH, CH)])


def _make_agg_kernel(ncols, gather_table):
    mesh = plsc.VectorSubcoreMesh(core_axis_name="c", subcore_axis_name="s")

    def body(*refs):
        if gather_table:
            table_hbm, src_hbm, dst_hbm, out_hbm, idx_v, rows_v, acc_sh, sem = refs
        else:
            dst_hbm, out_hbm, idx_v, rows_v, acc_sh, sem = refs
        cid = lax.axis_index("c")
        sid = lax.axis_index("s")

        _zero_vmem_2d(rows_v, CH, ncols, 0.0)
        zbase = sid * ROWS_PER_TILE
        for k in range(5):
            pltpu.sync_copy(rows_v, acc_sh.at[pl.ds(zbase + k * CH, CH)])
        if not gather_table:
            _zero_vmem_2d(rows_v, CH, ncols, 1.0)
        plsc.subcore_barrier()

        ebase = (cid * NSUB + sid) * EPT

        @pl.loop(0, NCH)
        def _(ch):
            off = ebase + ch * CH
            if gather_table:
                pltpu.sync_copy(src_hbm.at[pl.ds(off, CH)], idx_v.at[0])
                pltpu.sync_copy(dst_hbm.at[pl.ds(off, CH)], idx_v.at[1])
                pltpu.async_copy(table_hbm.at[idx_v.at[0]], rows_v, sem).wait()
            else:
                pltpu.sync_copy(dst_hbm.at[pl.ds(off, CH)], idx_v.at[1])
            pltpu.sync_copy(rows_v, acc_sh.at[idx_v.at[1]], add=True)

        plsc.subcore_barrier()
        _copy_acc_out(acc_sh, out_hbm, cid, sid, rows_v)

    return pl.kernel(
        body,
        out_type=jax.ShapeDtypeStruct((NCORES, NPAD, ncols), jnp.float32),
        mesh=mesh,
        scratch_types=[
            pltpu.VMEM((2, CH), jnp.int32),
            pltpu.VMEM((CH, ncols), jnp.float32),
            pltpu.VMEM_SHARED((NPAD, ncols), jnp.float32),
            pltpu.SemaphoreType.DMA,
        ],
    )


RB = 2560
GRID = NPAD // RB


def _dinv_scale_body(dp0_ref, dp1_ref, h_ref, dinv_ref, t_ref):
    i = pl.program_id(0)
    deg = dp0_ref[:, :1] + dp1_ref[:, :1] + 1.0
    row = lax.broadcasted_iota(jnp.int32, (RB, 1), 0) + i * RB
    dinv = jnp.where(row < N, lax.rsqrt(deg), 0.0)
    dinv_ref[...] = dinv
    t_ref[...] = h_ref[...] * dinv


def _matmul_body(x_ref, w_ref, o_ref):
    o_ref[...] = jnp.dot(x_ref[...], w_ref[...], preferred_element_type=jnp.float32)


def _combine_body(a0_ref, a1_ref, t_ref, dinv_ref, b_ref, w_ref, o_ref, *, relu_next):
    dinv = dinv_ref[...]
    u = (a0_ref[0] + a1_ref[0] + t_ref[...]) * dinv + b_ref[...]
    if relu_next:
        u = jnp.maximum(u, 0.0)
        o_ref[...] = jnp.dot(u, w_ref[...], preferred_element_type=jnp.float32) * dinv
    else:
        o_ref[...] = u


def _row_spec():
    return pl.BlockSpec((RB, D), lambda i: (i, 0))


def _tc_matmul(x, w):
    return pl.pallas_call(
        _matmul_body,
        grid=(GRID,),
        in_specs=[_row_spec(), pl.BlockSpec((D, D), lambda i: (0, 0))],
        out_specs=_row_spec(),
        out_shape=jax.ShapeDtypeStruct((NPAD, D), jnp.float32),
    )(x, w)


def _tc_dinv_scale(dp0, dp1, h):
    return pl.pallas_call(
        _dinv_scale_body,
        grid=(GRID,),
        in_specs=[
            pl.BlockSpec((RB, 16), lambda i: (i, 0)),
            pl.BlockSpec((RB, 16), lambda i: (i, 0)),
            _row_spec(),
        ],
        out_specs=[pl.BlockSpec((RB, 1), lambda i: (i, 0)), _row_spec()],
        out_shape=[
            jax.ShapeDtypeStruct((NPAD, 1), jnp.float32),
            jax.ShapeDtypeStruct((NPAD, D), jnp.float32),
        ],
    )(dp0, dp1, h)


def _tc_combine(a, t, dinv, b2d, w, relu_next):
    return pl.pallas_call(
        functools.partial(_combine_body, relu_next=relu_next),
        grid=(GRID,),
        in_specs=[
            pl.BlockSpec((1, RB, D), lambda i: (0, i, 0)),
            pl.BlockSpec((1, RB, D), lambda i: (1, i, 0)),
            _row_spec(),
            pl.BlockSpec((RB, 1), lambda i: (i, 0)),
            pl.BlockSpec((1, D), lambda i: (0, 0)),
            pl.BlockSpec((D, D), lambda i: (0, 0)),
        ],
        out_specs=_row_spec(),
        out_shape=jax.ShapeDtypeStruct((NPAD, D), jnp.float32),
    )(a, a, t, dinv, b2d, w)


_agg_rows = _make_agg_kernel(D, gather_table=True)
_agg_deg = _make_agg_kernel(16, gather_table=False)


@jax.jit
def kernel(x, edge_index, W1, b1, W2, b2, W3, b3):
    src = edge_index[0].astype(jnp.int32)
    dst = edge_index[1].astype(jnp.int32)
    padi = N + jnp.arange(EP - E, dtype=jnp.int32) % (NPAD - N)
    src_p = jnp.concatenate([src, padi])
    dst_p = jnp.concatenate([dst, padi])

    x_p = jnp.zeros((NPAD, D), jnp.float32).at[:N].set(x)
    b1r = b1.reshape(1, D)
    b2r = b2.reshape(1, D)
    b3r = b3.reshape(1, D)

    degp = _agg_deg(dst_p)
    h1 = _tc_matmul(x_p, W1)
    dinv, t1 = _tc_dinv_scale(degp[0], degp[1], h1)

    a1 = _agg_rows(t1, src_p, dst_p)
    t2 = _tc_combine(a1, t1, dinv, b1r, W2, relu_next=True)

    a2 = _agg_rows(t2, src_p, dst_p)
    t3 = _tc_combine(a2, t2, dinv, b2r, W3, relu_next=True)

    a3 = _agg_rows(t3, src_p, dst_p)
    out = _tc_combine(a3, t3, dinv, b3r, W3, relu_next=False)
    return out[:N]


# speedup vs baseline: 20.6964x; 1.5008x over previous
import functools
import jax
import jax.numpy as jnp
from jax import lax
from jax.experimental import pallas as pl
from jax.experimental.pallas import tpu as pltpu
from jax.experimental.pallas import tpu_sc as plsc

N = 10000
E = 320000
D = 128

NPAD = 10240
EP = 327680
NCORES = 2
NSUB = 16
NTILES = NCORES * NSUB
EPT = EP // NTILES
CH = 128
NCH = EPT // CH
ROWS_PER_TILE = NPAD // NSUB


def _zero_vmem_2d(ref, nrows, ncols, val):
    v = jnp.full((16,), val, jnp.float32)

    @pl.loop(0, nrows)
    def _(r):
        @pl.loop(0, ncols // 16)
        def _(c):
            ref[r, pl.ds(c * 16, 16)] = v


def _copy_acc_out(acc_sh, out_hbm, cid, sid, rows_v):
    base = sid * ROWS_PER_TILE
    for k in range(5):
        pltpu.sync_copy(acc_sh.at[pl.ds(base + k * CH, CH)], rows_v)
        pltpu.sync_copy(rows_v, out_hbm.at[cid, pl.ds(base + k * CH, CH)])


def _make_agg_kernel(ncols, gather_table):
    mesh = plsc.VectorSubcoreMesh(core_axis_name="c", subcore_axis_name="s")

    def body(*refs):
        if gather_table:
            (table_hbm, src_hbm, dst_hbm, out_hbm,
             sidx2, didx2, rows_v, acc_sh, sem0, sem1) = refs
            sems = (sem0, sem1)
            sidx = (sidx2.at[0], sidx2.at[1])
            didx = (didx2.at[0], didx2.at[1])
        else:
            dst_hbm, out_hbm, didx2, rows_v, acc_sh = refs
            didx0 = didx2.at[0]
        cid = lax.axis_index("c")
        sid = lax.axis_index("s")
        ebase = (cid * NSUB + sid) * EPT

        _zero_vmem_2d(rows_v.at[0], CH, ncols, 0.0)
        zbase = sid * ROWS_PER_TILE
        for k in range(5):
            pltpu.sync_copy(rows_v.at[0], acc_sh.at[pl.ds(zbase + k * CH, CH)])
        if not gather_table:
            _zero_vmem_2d(rows_v.at[0], CH, ncols, 1.0)
        plsc.subcore_barrier()

        if gather_table:
            def load_idx(c, b):
                off = ebase + c * CH
                pltpu.sync_copy(src_hbm.at[pl.ds(off, CH)], sidx[b])
                pltpu.sync_copy(dst_hbm.at[pl.ds(off, CH)], didx[b])

            def gather_start(b):
                pltpu.async_copy(table_hbm.at[sidx[b]], rows_v.at[b], sems[b])

            def gather_wait(b):
                pltpu.make_async_copy(
                    table_hbm.at[sidx[b]], rows_v.at[b], sems[b]).wait()

            load_idx(0, 0)
            gather_start(0)

            @pl.loop(0, NCH, step=2)
            def _(ch):
                for b in range(2):
                    c = ch + b
                    nb = 1 - b

                    @pl.when(c + 1 < NCH)
                    def _():
                        load_idx(c + 1, nb)
                    gather_wait(b)

                    @pl.when(c + 1 < NCH)
                    def _():
                        gather_start(nb)
                    pltpu.sync_copy(rows_v.at[b], acc_sh.at[didx[b]], add=True)
        else:
            @pl.loop(0, NCH)
            def _(ch):
                pltpu.sync_copy(dst_hbm.at[pl.ds(ebase + ch * CH, CH)], didx0)
                pltpu.sync_copy(rows_v.at[0], acc_sh.at[didx0], add=True)

        plsc.subcore_barrier()
        _copy_acc_out(acc_sh, out_hbm, cid, sid, rows_v.at[0])

    scratch = [
        pltpu.VMEM((2, CH), jnp.int32),
        pltpu.VMEM((2, CH, ncols), jnp.float32),
        pltpu.VMEM_SHARED((NPAD, ncols), jnp.float32),
    ]
    if gather_table:
        scratch = [pltpu.VMEM((2, CH), jnp.int32)] + scratch + [
            pltpu.SemaphoreType.DMA, pltpu.SemaphoreType.DMA]
    return pl.kernel(
        body,
        out_type=jax.ShapeDtypeStruct((NCORES, NPAD, ncols), jnp.float32),
        mesh=mesh,
        scratch_types=scratch,
    )


RB = 2560
GRID = NPAD // RB


def _dinv_scale_body(dp0_ref, dp1_ref, h_ref, dinv_ref, t_ref):
    i = pl.program_id(0)
    deg = dp0_ref[:, :1] + dp1_ref[:, :1] + 1.0
    row = lax.broadcasted_iota(jnp.int32, (RB, 1), 0) + i * RB
    dinv = jnp.where(row < N, lax.rsqrt(deg), 0.0)
    dinv_ref[...] = dinv
    t_ref[...] = h_ref[...] * dinv


def _matmul_body(x_ref, w_ref, o_ref):
    o_ref[...] = jnp.dot(x_ref[...], w_ref[...], preferred_element_type=jnp.float32)


def _combine_body(a0_ref, a1_ref, t_ref, dinv_ref, b_ref, w_ref, o_ref, *, relu_next):
    dinv = dinv_ref[...]
    u = (a0_ref[0] + a1_ref[0] + t_ref[...]) * dinv + b_ref[...]
    if relu_next:
        u = jnp.maximum(u, 0.0)
        o_ref[...] = jnp.dot(u, w_ref[...], preferred_element_type=jnp.float32) * dinv
    else:
        o_ref[...] = u


def _row_spec():
    return pl.BlockSpec((RB, D), lambda i: (i, 0))


def _tc_matmul(x, w):
    return pl.pallas_call(
        _matmul_body,
        grid=(GRID,),
        in_specs=[_row_spec(), pl.BlockSpec((D, D), lambda i: (0, 0))],
        out_specs=_row_spec(),
        out_shape=jax.ShapeDtypeStruct((NPAD, D), jnp.float32),
    )(x, w)


def _tc_dinv_scale(dp0, dp1, h):
    return pl.pallas_call(
        _dinv_scale_body,
        grid=(GRID,),
        in_specs=[
            pl.BlockSpec((RB, 16), lambda i: (i, 0)),
            pl.BlockSpec((RB, 16), lambda i: (i, 0)),
            _row_spec(),
        ],
        out_specs=[pl.BlockSpec((RB, 1), lambda i: (i, 0)), _row_spec()],
        out_shape=[
            jax.ShapeDtypeStruct((NPAD, 1), jnp.float32),
            jax.ShapeDtypeStruct((NPAD, D), jnp.float32),
        ],
    )(dp0, dp1, h)


def _tc_combine(a, t, dinv, b2d, w, relu_next):
    return pl.pallas_call(
        functools.partial(_combine_body, relu_next=relu_next),
        grid=(GRID,),
        in_specs=[
            pl.BlockSpec((1, RB, D), lambda i: (0, i, 0)),
            pl.BlockSpec((1, RB, D), lambda i: (1, i, 0)),
            _row_spec(),
            pl.BlockSpec((RB, 1), lambda i: (i, 0)),
            pl.BlockSpec((1, D), lambda i: (0, 0)),
            pl.BlockSpec((D, D), lambda i: (0, 0)),
        ],
        out_specs=_row_spec(),
        out_shape=jax.ShapeDtypeStruct((NPAD, D), jnp.float32),
    )(a, a, t, dinv, b2d, w)


_agg_rows = _make_agg_kernel(D, gather_table=True)
_agg_deg = _make_agg_kernel(16, gather_table=False)


@jax.jit
def kernel(x, edge_index, W1, b1, W2, b2, W3, b3):
    src = edge_index[0].astype(jnp.int32)
    dst = edge_index[1].astype(jnp.int32)
    padi = N + jnp.arange(EP - E, dtype=jnp.int32) % (NPAD - N)
    src_p = jnp.concatenate([src, padi])
    dst_p = jnp.concatenate([dst, padi])

    x_p = jnp.zeros((NPAD, D), jnp.float32).at[:N].set(x)
    b1r = b1.reshape(1, D)
    b2r = b2.reshape(1, D)
    b3r = b3.reshape(1, D)

    degp = _agg_deg(dst_p)
    h1 = _tc_matmul(x_p, W1)
    dinv, t1 = _tc_dinv_scale(degp[0], degp[1], h1)

    a1 = _agg_rows(t1, src_p, dst_p)
    t2 = _tc_combine(a1, t1, dinv, b1r, W2, relu_next=True)

    a2 = _agg_rows(t2, src_p, dst_p)
    t3 = _tc_combine(a2, t2, dinv, b2r, W3, relu_next=True)

    a3 = _agg_rows(t3, src_p, dst_p)
    out = _tc_combine(a3, t3, dinv, b3r, W3, relu_next=False)
    return out[:N]


# speedup vs baseline: 22.9899x; 1.1108x over previous
import functools
import jax
import jax.numpy as jnp
from jax import lax
from jax.experimental import pallas as pl
from jax.experimental.pallas import tpu as pltpu
from jax.experimental.pallas import tpu_sc as plsc

N = 10000
E = 320000
D = 128

NPAD = 10240
EP = 327680
NCORES = 2
NSUB = 16
NTILES = NCORES * NSUB
EPT = EP // NTILES
CH = 128
NCH = EPT // CH
ROWS_PER_TILE = NPAD // NSUB


def _zero_vmem_2d(ref, nrows, ncols, val):
    v = jnp.full((16,), val, jnp.float32)

    @pl.loop(0, nrows)
    def _(r):
        @pl.loop(0, ncols // 16)
        def _(c):
            ref[r, pl.ds(c * 16, 16)] = v


def _copy_acc_out(acc_sh, out_hbm, cid, sid, rows_v):
    base = sid * ROWS_PER_TILE
    for k in range(5):
        pltpu.sync_copy(acc_sh.at[pl.ds(base + k * CH, CH)], rows_v)
        pltpu.sync_copy(rows_v, out_hbm.at[cid, pl.ds(base + k * CH, CH)])


def _make_agg_kernel(ncols, gather_table):
    mesh = plsc.VectorSubcoreMesh(core_axis_name="c", subcore_axis_name="s")

    def body(*refs):
        if gather_table:
            (table_hbm, src_hbm, dst_hbm, out_hbm,
             sidx2, didx2, rows_v, acc_sh, sem0, sem1, isem0, isem1) = refs
            sems = (sem0, sem1)
            isems = (isem0, isem1)
            sidx = (sidx2.at[0], sidx2.at[1])
            didx = (didx2.at[0], didx2.at[1])
        else:
            dst_hbm, out_hbm, didx2, rows_v, acc_sh = refs
            didx0 = didx2.at[0]
        cid = lax.axis_index("c")
        sid = lax.axis_index("s")
        ebase = (cid * NSUB + sid) * EPT

        _zero_vmem_2d(rows_v.at[0], CH, ncols, 0.0)
        zbase = sid * ROWS_PER_TILE
        for k in range(5):
            pltpu.sync_copy(rows_v.at[0], acc_sh.at[pl.ds(zbase + k * CH, CH)])
        if not gather_table:
            _zero_vmem_2d(rows_v.at[0], CH, ncols, 1.0)
        plsc.subcore_barrier()

        if gather_table:
            def idx_start(c, b):
                off = ebase + c * CH
                pltpu.async_copy(src_hbm.at[pl.ds(off, CH)], sidx[b], isems[b])
                pltpu.async_copy(dst_hbm.at[pl.ds(off, CH)], didx[b], isems[b])

            def idx_wait(c, b):
                off = ebase + c * CH
                pltpu.make_async_copy(src_hbm.at[pl.ds(off, CH)], sidx[b],
                                      isems[b]).wait()
                pltpu.make_async_copy(dst_hbm.at[pl.ds(off, CH)], didx[b],
                                      isems[b]).wait()

            def gather_start(b):
                pltpu.async_copy(table_hbm.at[sidx[b]], rows_v.at[b], sems[b])

            def gather_wait(b):
                pltpu.make_async_copy(
                    table_hbm.at[sidx[b]], rows_v.at[b], sems[b]).wait()

            idx_start(0, 0)
            idx_wait(0, 0)
            gather_start(0)
            idx_start(1, 1)

            @pl.loop(0, NCH, step=2)
            def _(ch):
                for b in range(2):
                    c = ch + b
                    nb = 1 - b
                    gather_wait(b)

                    @pl.when(c + 1 < NCH)
                    def _():
                        idx_wait(c + 1, nb)
                        gather_start(nb)
                    pltpu.sync_copy(rows_v.at[b], acc_sh.at[didx[b]], add=True)

                    @pl.when(c + 2 < NCH)
                    def _():
                        idx_start(c + 2, b)
        else:
            @pl.loop(0, NCH)
            def _(ch):
                pltpu.sync_copy(dst_hbm.at[pl.ds(ebase + ch * CH, CH)], didx0)
                pltpu.sync_copy(rows_v.at[0], acc_sh.at[didx0], add=True)

        plsc.subcore_barrier()
        _copy_acc_out(acc_sh, out_hbm, cid, sid, rows_v.at[0])

    scratch = [
        pltpu.VMEM((2, CH), jnp.int32),
        pltpu.VMEM((2, CH, ncols), jnp.float32),
        pltpu.VMEM_SHARED((NPAD, ncols), jnp.float32),
    ]
    if gather_table:
        scratch = [pltpu.VMEM((2, CH), jnp.int32)] + scratch + [
            pltpu.SemaphoreType.DMA, pltpu.SemaphoreType.DMA,
            pltpu.SemaphoreType.DMA, pltpu.SemaphoreType.DMA]
    return pl.kernel(
        body,
        out_type=jax.ShapeDtypeStruct((NCORES, NPAD, ncols), jnp.float32),
        mesh=mesh,
        scratch_types=scratch,
    )


RB = 2560
GRID = NPAD // RB


def _dinv_scale_body(dp0_ref, dp1_ref, h_ref, dinv_ref, t_ref):
    i = pl.program_id(0)
    deg = dp0_ref[:, :1] + dp1_ref[:, :1] + 1.0
    row = lax.broadcasted_iota(jnp.int32, (RB, 1), 0) + i * RB
    dinv = jnp.where(row < N, lax.rsqrt(deg), 0.0)
    dinv_ref[...] = dinv
    t_ref[...] = h_ref[...] * dinv


def _matmul_body(x_ref, w_ref, o_ref):
    o_ref[...] = jnp.dot(x_ref[...], w_ref[...], preferred_element_type=jnp.float32)


def _combine_body(a0_ref, a1_ref, t_ref, dinv_ref, b_ref, w_ref, o_ref, *, relu_next):
    dinv = dinv_ref[...]
    u = (a0_ref[0] + a1_ref[0] + t_ref[...]) * dinv + b_ref[...]
    if relu_next:
        u = jnp.maximum(u, 0.0)
        o_ref[...] = jnp.dot(u, w_ref[...], preferred_element_type=jnp.float32) * dinv
    else:
        o_ref[...] = u


def _row_spec():
    return pl.BlockSpec((RB, D), lambda i: (i, 0))


def _tc_matmul(x, w):
    return pl.pallas_call(
        _matmul_body,
        grid=(GRID,),
        in_specs=[_row_spec(), pl.BlockSpec((D, D), lambda i: (0, 0))],
        out_specs=_row_spec(),
        out_shape=jax.ShapeDtypeStruct((NPAD, D), jnp.float32),
    )(x, w)


def _tc_dinv_scale(dp0, dp1, h):
    return pl.pallas_call(
        _dinv_scale_body,
        grid=(GRID,),
        in_specs=[
            pl.BlockSpec((RB, 16), lambda i: (i, 0)),
            pl.BlockSpec((RB, 16), lambda i: (i, 0)),
            _row_spec(),
        ],
        out_specs=[pl.BlockSpec((RB, 1), lambda i: (i, 0)), _row_spec()],
        out_shape=[
            jax.ShapeDtypeStruct((NPAD, 1), jnp.float32),
            jax.ShapeDtypeStruct((NPAD, D), jnp.float32),
        ],
    )(dp0, dp1, h)


def _tc_combine(a, t, dinv, b2d, w, relu_next):
    return pl.pallas_call(
        functools.partial(_combine_body, relu_next=relu_next),
        grid=(GRID,),
        in_specs=[
            pl.BlockSpec((1, RB, D), lambda i: (0, i, 0)),
            pl.BlockSpec((1, RB, D), lambda i: (1, i, 0)),
            _row_spec(),
            pl.BlockSpec((RB, 1), lambda i: (i, 0)),
            pl.BlockSpec((1, D), lambda i: (0, 0)),
            pl.BlockSpec((D, D), lambda i: (0, 0)),
        ],
        out_specs=_row_spec(),
        out_shape=jax.ShapeDtypeStruct((NPAD, D), jnp.float32),
    )(a, a, t, dinv, b2d, w)


_agg_rows = _make_agg_kernel(D, gather_table=True)
_agg_deg = _make_agg_kernel(16, gather_table=False)


@jax.jit
def kernel(x, edge_index, W1, b1, W2, b2, W3, b3):
    src = edge_index[0].astype(jnp.int32)
    dst = edge_index[1].astype(jnp.int32)
    padi = N + jnp.arange(EP - E, dtype=jnp.int32) % (NPAD - N)
    src_p = jnp.concatenate([src, padi])
    dst_p = jnp.concatenate([dst, padi])

    x_p = jnp.zeros((NPAD, D), jnp.float32).at[:N].set(x)
    b1r = b1.reshape(1, D)
    b2r = b2.reshape(1, D)
    b3r = b3.reshape(1, D)

    degp = _agg_deg(dst_p)
    h1 = _tc_matmul(x_p, W1)
    dinv, t1 = _tc_dinv_scale(degp[0], degp[1], h1)

    a1 = _agg_rows(t1, src_p, dst_p)
    t2 = _tc_combine(a1, t1, dinv, b1r, W2, relu_next=True)

    a2 = _agg_rows(t2, src_p, dst_p)
    t3 = _tc_combine(a2, t2, dinv, b2r, W3, relu_next=True)

    a3 = _agg_rows(t3, src_p, dst_p)
    out = _tc_combine(a3, t3, dinv, b3r, W3, relu_next=False)
    return out[:N]


# speedup vs baseline: 24.3557x; 1.0594x over previous
import functools
import jax
import jax.numpy as jnp
from jax import lax
from jax.experimental import pallas as pl
from jax.experimental.pallas import tpu as pltpu
from jax.experimental.pallas import tpu_sc as plsc

N = 10000
E = 320000
D = 128

NPAD = 10240
EP = 327680
NCORES = 2
NSUB = 16
NTILES = NCORES * NSUB
EPT = EP // NTILES
CH = 128
NCH = EPT // CH
ROWS_PER_TILE = NPAD // NSUB


def _zero_vmem_2d(ref, nrows, ncols, val):
    v = jnp.full((16,), val, jnp.float32)

    @pl.loop(0, nrows)
    def _(r):
        @pl.loop(0, ncols // 16)
        def _(c):
            ref[r, pl.ds(c * 16, 16)] = v


def _copy_acc_out(acc_sh, out_hbm, cid, sid, rows_v):
    base = sid * ROWS_PER_TILE
    for k in range(5):
        pltpu.sync_copy(acc_sh.at[pl.ds(base + k * CH, CH)], rows_v)
        pltpu.sync_copy(rows_v, out_hbm.at[cid, pl.ds(base + k * CH, CH)])


def _make_agg_kernel(ncols, gather_table):
    mesh = plsc.VectorSubcoreMesh(core_axis_name="c", subcore_axis_name="s")

    def body(*refs):
        if gather_table:
            (table_hbm, src_hbm, dst_hbm, out_hbm,
             sidx4, didx4, rows_v, acc_sh, sem0, sem1, ssem0, ssem1,
             isem0, isem1, isem2, isem3) = refs
            sems = (sem0, sem1)
            ssems = (ssem0, ssem1)
            sidx = tuple(sidx4.at[s] for s in range(4))
        else:
            (dst_hbm, out_hbm, didx4, rows_v, acc_sh,
             ssem0, isem0, isem1, isem2, isem3) = refs
            ssems = (ssem0, ssem0)
        didx = tuple(didx4.at[s] for s in range(4))
        isems = (isem0, isem1, isem2, isem3)
        cid = lax.axis_index("c")
        sid = lax.axis_index("s")
        ebase = (cid * NSUB + sid) * EPT

        _zero_vmem_2d(rows_v.at[0], CH, ncols, 0.0)
        zbase = sid * ROWS_PER_TILE
        for k in range(5):
            pltpu.sync_copy(rows_v.at[0], acc_sh.at[pl.ds(zbase + k * CH, CH)])
        if not gather_table:
            _zero_vmem_2d(rows_v.at[0], CH, ncols, 1.0)
        plsc.subcore_barrier()

        def idx_start(c, s):
            off = ebase + c * CH
            if gather_table:
                pltpu.async_copy(src_hbm.at[pl.ds(off, CH)], sidx[s], isems[s])
            pltpu.async_copy(dst_hbm.at[pl.ds(off, CH)], didx[s], isems[s])

        def idx_wait(c, s):
            off = ebase + c * CH
            if gather_table:
                pltpu.make_async_copy(src_hbm.at[pl.ds(off, CH)], sidx[s],
                                      isems[s]).wait()
            pltpu.make_async_copy(dst_hbm.at[pl.ds(off, CH)], didx[s],
                                  isems[s]).wait()

        def gather_start(s, rb):
            pltpu.async_copy(table_hbm.at[sidx[s]], rows_v.at[rb], sems[rb])

        def gather_wait(s, rb):
            pltpu.make_async_copy(
                table_hbm.at[sidx[s]], rows_v.at[rb], sems[rb]).wait()

        def scatter_start(s, rb):
            pltpu.async_copy(rows_v.at[rb], acc_sh.at[didx[s]], ssems[rb],
                             add=True)

        def scatter_wait(s, rb):
            pltpu.make_async_copy(rows_v.at[rb], acc_sh.at[didx[s]],
                                  ssems[rb]).wait()

        if gather_table:
            idx_start(0, 0)
            idx_wait(0, 0)
            gather_start(0, 0)
            idx_start(1, 1)

            @pl.loop(0, NCH, step=4)
            def _(ch):
                for b in range(4):
                    c = ch + b
                    rb = b % 2
                    nrb = 1 - rb
                    s1, s2, sp = (b + 1) % 4, (b + 2) % 4, (b + 3) % 4
                    gather_wait(b, rb)

                    @pl.when(c >= 1)
                    def _():
                        scatter_wait(sp, nrb)

                    @pl.when(c + 1 < NCH)
                    def _():
                        idx_wait(c + 1, s1)
                        gather_start(s1, nrb)
                    scatter_start(b, rb)

                    @pl.when(c + 2 < NCH)
                    def _():
                        idx_start(c + 2, s2)

            scatter_wait(3, 1)
        else:
            idx_start(0, 0)
            idx_start(1, 1)

            @pl.loop(0, NCH, step=4)
            def _(ch):
                for b in range(4):
                    c = ch + b
                    s2, sp = (b + 2) % 4, (b + 3) % 4

                    @pl.when(c >= 1)
                    def _():
                        scatter_wait(sp, 0)
                    idx_wait(c, b)
                    scatter_start(b, 0)

                    @pl.when(c + 2 < NCH)
                    def _():
                        idx_start(c + 2, s2)

            scatter_wait(3, 0)

        plsc.subcore_barrier()
        _copy_acc_out(acc_sh, out_hbm, cid, sid, rows_v.at[0])

    scratch = [
        pltpu.VMEM((4, CH), jnp.int32),
        pltpu.VMEM((2, CH, ncols), jnp.float32),
        pltpu.VMEM_SHARED((NPAD, ncols), jnp.float32),
    ]
    nsem = 8 if gather_table else 5
    if gather_table:
        scratch = [pltpu.VMEM((4, CH), jnp.int32)] + scratch
    scratch += [pltpu.SemaphoreType.DMA] * nsem
    return pl.kernel(
        body,
        out_type=jax.ShapeDtypeStruct((NCORES, NPAD, ncols), jnp.float32),
        mesh=mesh,
        scratch_types=scratch,
    )


RB = 2560
GRID = NPAD // RB


def _dinv_scale_body(dp0_ref, dp1_ref, h_ref, dinv_ref, t_ref):
    i = pl.program_id(0)
    deg = dp0_ref[:, :1] + dp1_ref[:, :1] + 1.0
    row = lax.broadcasted_iota(jnp.int32, (RB, 1), 0) + i * RB
    dinv = jnp.where(row < N, lax.rsqrt(deg), 0.0)
    dinv_ref[...] = dinv
    t_ref[...] = h_ref[...] * dinv


def _matmul_body(x_ref, w_ref, o_ref):
    o_ref[...] = jnp.dot(x_ref[...], w_ref[...], preferred_element_type=jnp.float32)


def _combine_body(a0_ref, a1_ref, t_ref, dinv_ref, b_ref, w_ref, o_ref, *, relu_next):
    dinv = dinv_ref[...]
    u = (a0_ref[0] + a1_ref[0] + t_ref[...]) * dinv + b_ref[...]
    if relu_next:
        u = jnp.maximum(u, 0.0)
        o_ref[...] = jnp.dot(u, w_ref[...], preferred_element_type=jnp.float32) * dinv
    else:
        o_ref[...] = u


def _row_spec():
    return pl.BlockSpec((RB, D), lambda i: (i, 0))


def _tc_matmul(x, w):
    return pl.pallas_call(
        _matmul_body,
        grid=(GRID,),
        in_specs=[_row_spec(), pl.BlockSpec((D, D), lambda i: (0, 0))],
        out_specs=_row_spec(),
        out_shape=jax.ShapeDtypeStruct((NPAD, D), jnp.float32),
    )(x, w)


def _tc_dinv_scale(dp0, dp1, h):
    return pl.pallas_call(
        _dinv_scale_body,
        grid=(GRID,),
        in_specs=[
            pl.BlockSpec((RB, 16), lambda i: (i, 0)),
            pl.BlockSpec((RB, 16), lambda i: (i, 0)),
            _row_spec(),
        ],
        out_specs=[pl.BlockSpec((RB, 1), lambda i: (i, 0)), _row_spec()],
        out_shape=[
            jax.ShapeDtypeStruct((NPAD, 1), jnp.float32),
            jax.ShapeDtypeStruct((NPAD, D), jnp.float32),
        ],
    )(dp0, dp1, h)


def _tc_combine(a, t, dinv, b2d, w, relu_next):
    return pl.pallas_call(
        functools.partial(_combine_body, relu_next=relu_next),
        grid=(GRID,),
        in_specs=[
            pl.BlockSpec((1, RB, D), lambda i: (0, i, 0)),
            pl.BlockSpec((1, RB, D), lambda i: (1, i, 0)),
            _row_spec(),
            pl.BlockSpec((RB, 1), lambda i: (i, 0)),
            pl.BlockSpec((1, D), lambda i: (0, 0)),
            pl.BlockSpec((D, D), lambda i: (0, 0)),
        ],
        out_specs=_row_spec(),
        out_shape=jax.ShapeDtypeStruct((NPAD, D), jnp.float32),
    )(a, a, t, dinv, b2d, w)


_agg_rows = _make_agg_kernel(D, gather_table=True)
_agg_deg = _make_agg_kernel(16, gather_table=False)


@jax.jit
def kernel(x, edge_index, W1, b1, W2, b2, W3, b3):
    src = edge_index[0].astype(jnp.int32)
    dst = edge_index[1].astype(jnp.int32)
    padi = N + jnp.arange(EP - E, dtype=jnp.int32) % (NPAD - N)
    src_p = jnp.concatenate([src, padi])
    dst_p = jnp.concatenate([dst, padi])

    x_p = jnp.zeros((NPAD, D), jnp.float32).at[:N].set(x)
    b1r = b1.reshape(1, D)
    b2r = b2.reshape(1, D)
    b3r = b3.reshape(1, D)

    degp = _agg_deg(dst_p)
    h1 = _tc_matmul(x_p, W1)
    dinv, t1 = _tc_dinv_scale(degp[0], degp[1], h1)

    a1 = _agg_rows(t1, src_p, dst_p)
    t2 = _tc_combine(a1, t1, dinv, b1r, W2, relu_next=True)

    a2 = _agg_rows(t2, src_p, dst_p)
    t3 = _tc_combine(a2, t2, dinv, b2r, W3, relu_next=True)

    a3 = _agg_rows(t3, src_p, dst_p)
    out = _tc_combine(a3, t3, dinv, b3r, W3, relu_next=False)
    return out[:N]


# speedup vs baseline: 24.3613x; 1.0002x over previous
import functools
import jax
import jax.numpy as jnp
from jax import lax
from jax.experimental import pallas as pl
from jax.experimental.pallas import tpu as pltpu
from jax.experimental.pallas import tpu_sc as plsc

N = 10000
E = 320000
D = 128

NPAD = 10240
EP = 327680
NCORES = 2
NSUB = 16
NTILES = NCORES * NSUB
EPT = EP // NTILES
CH = 128
NCH = EPT // CH
ROWS_PER_TILE = NPAD // NSUB


def _zero_vmem_2d(ref, nrows, ncols, val):
    v = jnp.full((16,), val, jnp.float32)

    @pl.loop(0, nrows)
    def _(r):
        @pl.loop(0, ncols // 16)
        def _(c):
            ref[r, pl.ds(c * 16, 16)] = v


def _copy_acc_out(acc_sh, out_hbm, cid, sid, rows_v):
    base = sid * ROWS_PER_TILE
    for k in range(5):
        pltpu.sync_copy(acc_sh.at[pl.ds(base + k * CH, CH)], rows_v)
        pltpu.sync_copy(rows_v, out_hbm.at[cid, pl.ds(base + k * CH, CH)])


def _make_agg_kernel(ncols, gather_table):
    mesh = plsc.VectorSubcoreMesh(core_axis_name="c", subcore_axis_name="s")

    def body(*refs):
        if gather_table:
            (table_hbm, src_hbm, dst_hbm, out_hbm,
             sidx4, didx4, rows_v, acc_sh, sem0, sem1, ssem0, ssem1,
             isem0, isem1, isem2, isem3) = refs
            sems = (sem0, sem1)
            ssems = (ssem0, ssem1)
            sidx = tuple(sidx4.at[s] for s in range(4))
        else:
            (dst_hbm, out_hbm, didx4, rows_v, acc_sh,
             ssem0, isem0, isem1, isem2, isem3) = refs
            ssems = (ssem0, ssem0)
        didx = tuple(didx4.at[s] for s in range(4))
        isems = (isem0, isem1, isem2, isem3)
        cid = lax.axis_index("c")
        sid = lax.axis_index("s")
        ebase = (cid * NSUB + sid) * EPT

        _zero_vmem_2d(rows_v.at[0], CH, ncols, 0.0)
        zbase = sid * ROWS_PER_TILE
        for k in range(5):
            pltpu.sync_copy(rows_v.at[0], acc_sh.at[pl.ds(zbase + k * CH, CH)])
        if not gather_table:
            _zero_vmem_2d(rows_v.at[0], CH, ncols, 1.0)
        plsc.subcore_barrier()

        def idx_start(c, s):
            off = ebase + c * CH
            if gather_table:
                pltpu.async_copy(src_hbm.at[pl.ds(off, CH)], sidx[s], isems[s])
            pltpu.async_copy(dst_hbm.at[pl.ds(off, CH)], didx[s], isems[s])

        def idx_wait(c, s):
            off = ebase + c * CH
            if gather_table:
                pltpu.make_async_copy(src_hbm.at[pl.ds(off, CH)], sidx[s],
                                      isems[s]).wait()
            pltpu.make_async_copy(dst_hbm.at[pl.ds(off, CH)], didx[s],
                                  isems[s]).wait()

        def gather_start(s, rb):
            pltpu.async_copy(table_hbm.at[sidx[s]], rows_v.at[rb], sems[rb])

        def gather_wait(s, rb):
            pltpu.make_async_copy(
                table_hbm.at[sidx[s]], rows_v.at[rb], sems[rb]).wait()

        def scatter_start(s, rb):
            pltpu.async_copy(rows_v.at[rb], acc_sh.at[didx[s]], ssems[rb],
                             add=True)

        def scatter_wait(s, rb):
            pltpu.make_async_copy(rows_v.at[rb], acc_sh.at[didx[s]],
                                  ssems[rb]).wait()

        if gather_table:
            idx_start(0, 0)
            idx_wait(0, 0)
            gather_start(0, 0)
            idx_start(1, 1)

            @pl.loop(0, NCH, step=4)
            def _(ch):
                for b in range(4):
                    c = ch + b
                    rb = b % 2
                    nrb = 1 - rb
                    s1, s2, sp = (b + 1) % 4, (b + 2) % 4, (b + 3) % 4
                    gather_wait(b, rb)

                    @pl.when(c >= 1)
                    def _():
                        scatter_wait(sp, nrb)

                    @pl.when(c + 1 < NCH)
                    def _():
                        idx_wait(c + 1, s1)
                        gather_start(s1, nrb)
                    scatter_start(b, rb)

                    @pl.when(c + 2 < NCH)
                    def _():
                        idx_start(c + 2, s2)

            scatter_wait(3, 1)
        else:
            idx_start(0, 0)
            idx_start(1, 1)

            @pl.loop(0, NCH, step=4)
            def _(ch):
                for b in range(4):
                    c = ch + b
                    s2, sp = (b + 2) % 4, (b + 3) % 4

                    @pl.when(c >= 1)
                    def _():
                        scatter_wait(sp, 0)
                    idx_wait(c, b)
                    scatter_start(b, 0)

                    @pl.when(c + 2 < NCH)
                    def _():
                        idx_start(c + 2, s2)

            scatter_wait(3, 0)

        plsc.subcore_barrier()
        _copy_acc_out(acc_sh, out_hbm, cid, sid, rows_v.at[0])

    scratch = [
        pltpu.VMEM((4, CH), jnp.int32),
        pltpu.VMEM((2, CH, ncols), jnp.float32),
        pltpu.VMEM_SHARED((NPAD, ncols), jnp.float32),
    ]
    nsem = 8 if gather_table else 5
    if gather_table:
        scratch = [pltpu.VMEM((4, CH), jnp.int32)] + scratch
    scratch += [pltpu.SemaphoreType.DMA] * nsem
    return pl.kernel(
        body,
        out_type=jax.ShapeDtypeStruct((NCORES, NPAD, ncols), jnp.float32),
        mesh=mesh,
        scratch_types=scratch,
    )


RB = 2560
GRID = NPAD // RB


def _dinv_scale_body(dp0_ref, dp1_ref, x_ref, w_ref, dinv_ref, t_ref):
    i = pl.program_id(0)
    deg = dp0_ref[:, :1] + dp1_ref[:, :1] + 1.0
    row = lax.broadcasted_iota(jnp.int32, (RB, 1), 0) + i * RB
    dinv = jnp.where(row < N, lax.rsqrt(deg), 0.0)
    dinv_ref[...] = dinv
    h = jnp.dot(x_ref[...], w_ref[...], preferred_element_type=jnp.float32)
    t_ref[...] = h * dinv


def _matmul_body(x_ref, w_ref, o_ref):
    o_ref[...] = jnp.dot(x_ref[...], w_ref[...], preferred_element_type=jnp.float32)


def _combine_body(a0_ref, a1_ref, t_ref, dinv_ref, b_ref, w_ref, o_ref, *, relu_next):
    dinv = dinv_ref[...]
    u = (a0_ref[0] + a1_ref[0] + t_ref[...]) * dinv + b_ref[...]
    if relu_next:
        u = jnp.maximum(u, 0.0)
        o_ref[...] = jnp.dot(u, w_ref[...], preferred_element_type=jnp.float32) * dinv
    else:
        o_ref[...] = u


def _row_spec():
    return pl.BlockSpec((RB, D), lambda i: (i, 0))


def _tc_matmul(x, w):
    return pl.pallas_call(
        _matmul_body,
        grid=(GRID,),
        in_specs=[_row_spec(), pl.BlockSpec((D, D), lambda i: (0, 0))],
        out_specs=_row_spec(),
        out_shape=jax.ShapeDtypeStruct((NPAD, D), jnp.float32),
    )(x, w)


def _tc_dinv_scale(dp0, dp1, x, w):
    return pl.pallas_call(
        _dinv_scale_body,
        grid=(GRID,),
        in_specs=[
            pl.BlockSpec((RB, 16), lambda i: (i, 0)),
            pl.BlockSpec((RB, 16), lambda i: (i, 0)),
            _row_spec(),
            pl.BlockSpec((D, D), lambda i: (0, 0)),
        ],
        out_specs=[pl.BlockSpec((RB, 1), lambda i: (i, 0)), _row_spec()],
        out_shape=[
            jax.ShapeDtypeStruct((NPAD, 1), jnp.float32),
            jax.ShapeDtypeStruct((NPAD, D), jnp.float32),
        ],
    )(dp0, dp1, x, w)


def _tc_combine(a, t, dinv, b2d, w, relu_next):
    return pl.pallas_call(
        functools.partial(_combine_body, relu_next=relu_next),
        grid=(GRID,),
        in_specs=[
            pl.BlockSpec((1, RB, D), lambda i: (0, i, 0)),
            pl.BlockSpec((1, RB, D), lambda i: (1, i, 0)),
            _row_spec(),
            pl.BlockSpec((RB, 1), lambda i: (i, 0)),
            pl.BlockSpec((1, D), lambda i: (0, 0)),
            pl.BlockSpec((D, D), lambda i: (0, 0)),
        ],
        out_specs=_row_spec(),
        out_shape=jax.ShapeDtypeStruct((NPAD, D), jnp.float32),
    )(a, a, t, dinv, b2d, w)


_agg_rows = _make_agg_kernel(D, gather_table=True)
_agg_deg = _make_agg_kernel(16, gather_table=False)


@jax.jit
def kernel(x, edge_index, W1, b1, W2, b2, W3, b3):
    src = edge_index[0].astype(jnp.int32)
    dst = edge_index[1].astype(jnp.int32)
    padi = N + jnp.arange(EP - E, dtype=jnp.int32) % (NPAD - N)
    src_p = jnp.concatenate([src, padi])
    dst_p = jnp.concatenate([dst, padi])

    x_p = jnp.zeros((NPAD, D), jnp.float32).at[:N].set(x)
    b1r = b1.reshape(1, D)
    b2r = b2.reshape(1, D)
    b3r = b3.reshape(1, D)

    degp = _agg_deg(dst_p)
    dinv, t1 = _tc_dinv_scale(degp[0], degp[1], x_p, W1)

    a1 = _agg_rows(t1, src_p, dst_p)
    t2 = _tc_combine(a1, t1, dinv, b1r, W2, relu_next=True)

    a2 = _agg_rows(t2, src_p, dst_p)
    t3 = _tc_combine(a2, t2, dinv, b2r, W3, relu_next=True)

    a3 = _agg_rows(t3, src_p, dst_p)
    out = _tc_combine(a3, t3, dinv, b3r, W3, relu_next=False)
    return out[:N]


# speedup vs baseline: 26.9297x; 1.1054x over previous
import functools
import jax
import jax.numpy as jnp
from jax import lax
from jax.experimental import pallas as pl
from jax.experimental.pallas import tpu as pltpu
from jax.experimental.pallas import tpu_sc as plsc

N = 10000
E = 320000
D = 128

NPAD = 10240
EP = 327680
NCORES = 2
NSUB = 16
NTILES = NCORES * NSUB
EPT = EP // NTILES
CH = 64
NCH = EPT // CH
ROWS_PER_TILE = NPAD // NSUB


def _zero_vmem_2d(ref, nrows, ncols, val):
    v = jnp.full((16,), val, jnp.float32)

    @pl.loop(0, nrows)
    def _(r):
        @pl.loop(0, ncols // 16)
        def _(c):
            ref[r, pl.ds(c * 16, 16)] = v


def _copy_acc_out(acc_sh, out_hbm, cid, sid, rows_v):
    base = sid * ROWS_PER_TILE
    for k in range(ROWS_PER_TILE // CH):
        pltpu.sync_copy(acc_sh.at[pl.ds(base + k * CH, CH)], rows_v)
        pltpu.sync_copy(rows_v, out_hbm.at[cid, pl.ds(base + k * CH, CH)])


def _make_agg_kernel(ncols, gather_table):
    mesh = plsc.VectorSubcoreMesh(core_axis_name="c", subcore_axis_name="s")

    def body(*refs):
        if gather_table:
            (table_hbm, src_hbm, dst_hbm, out_hbm,
             sidx4, didx4, rows_v, acc_sh, sem0, sem1, sem2, sem3,
             ssem0, ssem1, ssem2, ssem3,
             isem0, isem1, isem2, isem3) = refs
            sems = (sem0, sem1, sem2, sem3)
            ssems = (ssem0, ssem1, ssem2, ssem3)
            sidx = tuple(sidx4.at[s] for s in range(4))
        else:
            (dst_hbm, out_hbm, didx4, rows_v, acc_sh,
             ssem0, isem0, isem1, isem2, isem3) = refs
            ssems = (ssem0, ssem0)
        didx = tuple(didx4.at[s] for s in range(4))
        isems = (isem0, isem1, isem2, isem3)
        cid = lax.axis_index("c")
        sid = lax.axis_index("s")
        ebase = (cid * NSUB + sid) * EPT

        _zero_vmem_2d(rows_v.at[0], CH, ncols, 0.0)
        zbase = sid * ROWS_PER_TILE
        for k in range(ROWS_PER_TILE // CH):
            pltpu.sync_copy(rows_v.at[0], acc_sh.at[pl.ds(zbase + k * CH, CH)])
        if not gather_table:
            _zero_vmem_2d(rows_v.at[0], CH, ncols, 1.0)
        plsc.subcore_barrier()

        def idx_start(c, s):
            off = ebase + c * CH
            if gather_table:
                pltpu.async_copy(src_hbm.at[pl.ds(off, CH)], sidx[s], isems[s])
            pltpu.async_copy(dst_hbm.at[pl.ds(off, CH)], didx[s], isems[s])

        def idx_wait(c, s):
            off = ebase + c * CH
            if gather_table:
                pltpu.make_async_copy(src_hbm.at[pl.ds(off, CH)], sidx[s],
                                      isems[s]).wait()
            pltpu.make_async_copy(dst_hbm.at[pl.ds(off, CH)], didx[s],
                                  isems[s]).wait()

        def gather_start(s, rb):
            pltpu.async_copy(table_hbm.at[sidx[s]], rows_v.at[rb], sems[rb])

        def gather_wait(s, rb):
            pltpu.make_async_copy(
                table_hbm.at[sidx[s]], rows_v.at[rb], sems[rb]).wait()

        def scatter_start(s, rb):
            pltpu.async_copy(rows_v.at[rb], acc_sh.at[didx[s]], ssems[rb],
                             add=True)

        def scatter_wait(s, rb):
            pltpu.make_async_copy(rows_v.at[rb], acc_sh.at[didx[s]],
                                  ssems[rb]).wait()

        if gather_table:
            idx_start(0, 0)
            idx_wait(0, 0)
            gather_start(0, 0)
            idx_start(1, 1)
            idx_wait(1, 1)
            gather_start(1, 1)
            idx_start(2, 2)

            @pl.loop(0, NCH, step=4)
            def _(ch):
                for b in range(4):
                    c = ch + b
                    s2, s3, sp = (b + 2) % 4, (b + 3) % 4, (b + 3) % 4
                    gather_wait(b, b)

                    @pl.when(c >= 1)
                    def _():
                        scatter_wait(sp, sp)

                    @pl.when(c + 2 < NCH)
                    def _():
                        idx_wait(c + 2, s2)
                        gather_start(s2, s2)
                    scatter_start(b, b)

                    @pl.when(c + 3 < NCH)
                    def _():
                        idx_start(c + 3, s3)

            scatter_wait((NCH - 1) % 4, (NCH - 1) % 4)
        else:
            idx_start(0, 0)
            idx_start(1, 1)

            @pl.loop(0, NCH, step=4)
            def _(ch):
                for b in range(4):
                    c = ch + b
                    s2, sp = (b + 2) % 4, (b + 3) % 4

                    @pl.when(c >= 1)
                    def _():
                        scatter_wait(sp, 0)
                    idx_wait(c, b)
                    scatter_start(b, 0)

                    @pl.when(c + 2 < NCH)
                    def _():
                        idx_start(c + 2, s2)

            scatter_wait(3, 0)

        plsc.subcore_barrier()
        _copy_acc_out(acc_sh, out_hbm, cid, sid, rows_v.at[0])

    scratch = [
        pltpu.VMEM((4, CH), jnp.int32),
        pltpu.VMEM((4, CH, ncols), jnp.float32),
        pltpu.VMEM_SHARED((NPAD, ncols), jnp.float32),
    ]
    nsem = 12 if gather_table else 5
    if gather_table:
        scratch = [pltpu.VMEM((4, CH), jnp.int32)] + scratch
    scratch += [pltpu.SemaphoreType.DMA] * nsem
    return pl.kernel(
        body,
        out_type=jax.ShapeDtypeStruct((NCORES, NPAD, ncols), jnp.float32),
        mesh=mesh,
        scratch_types=scratch,
    )


RB = 2560
GRID = NPAD // RB


def _dinv_scale_body(dp0_ref, dp1_ref, x_ref, w_ref, dinv_ref, t_ref):
    i = pl.program_id(0)
    deg = dp0_ref[:, :1] + dp1_ref[:, :1] + 1.0
    row = lax.broadcasted_iota(jnp.int32, (RB, 1), 0) + i * RB
    dinv = jnp.where(row < N, lax.rsqrt(deg), 0.0)
    dinv_ref[...] = dinv
    h = jnp.dot(x_ref[...], w_ref[...], preferred_element_type=jnp.float32)
    t_ref[...] = h * dinv


def _matmul_body(x_ref, w_ref, o_ref):
    o_ref[...] = jnp.dot(x_ref[...], w_ref[...], preferred_element_type=jnp.float32)


def _combine_body(a0_ref, a1_ref, t_ref, dinv_ref, b_ref, w_ref, o_ref, *, relu_next):
    dinv = dinv_ref[...]
    u = (a0_ref[0] + a1_ref[0] + t_ref[...]) * dinv + b_ref[...]
    if relu_next:
        u = jnp.maximum(u, 0.0)
        o_ref[...] = jnp.dot(u, w_ref[...], preferred_element_type=jnp.float32) * dinv
    else:
        o_ref[...] = u


def _row_spec():
    return pl.BlockSpec((RB, D), lambda i: (i, 0))


def _tc_matmul(x, w):
    return pl.pallas_call(
        _matmul_body,
        grid=(GRID,),
        in_specs=[_row_spec(), pl.BlockSpec((D, D), lambda i: (0, 0))],
        out_specs=_row_spec(),
        out_shape=jax.ShapeDtypeStruct((NPAD, D), jnp.float32),
    )(x, w)


def _tc_dinv_scale(dp0, dp1, x, w):
    return pl.pallas_call(
        _dinv_scale_body,
        grid=(GRID,),
        in_specs=[
            pl.BlockSpec((RB, 16), lambda i: (i, 0)),
            pl.BlockSpec((RB, 16), lambda i: (i, 0)),
            _row_spec(),
            pl.BlockSpec((D, D), lambda i: (0, 0)),
        ],
        out_specs=[pl.BlockSpec((RB, 1), lambda i: (i, 0)), _row_spec()],
        out_shape=[
            jax.ShapeDtypeStruct((NPAD, 1), jnp.float32),
            jax.ShapeDtypeStruct((NPAD, D), jnp.float32),
        ],
    )(dp0, dp1, x, w)


def _tc_combine(a, t, dinv, b2d, w, relu_next):
    return pl.pallas_call(
        functools.partial(_combine_body, relu_next=relu_next),
        grid=(GRID,),
        in_specs=[
            pl.BlockSpec((1, RB, D), lambda i: (0, i, 0)),
            pl.BlockSpec((1, RB, D), lambda i: (1, i, 0)),
            _row_spec(),
            pl.BlockSpec((RB, 1), lambda i: (i, 0)),
            pl.BlockSpec((1, D), lambda i: (0, 0)),
            pl.BlockSpec((D, D), lambda i: (0, 0)),
        ],
        out_specs=_row_spec(),
        out_shape=jax.ShapeDtypeStruct((NPAD, D), jnp.float32),
    )(a, a, t, dinv, b2d, w)


_agg_rows = _make_agg_kernel(D, gather_table=True)
_agg_deg = _make_agg_kernel(16, gather_table=False)


@jax.jit
def kernel(x, edge_index, W1, b1, W2, b2, W3, b3):
    src = edge_index[0].astype(jnp.int32)
    dst = edge_index[1].astype(jnp.int32)
    padi = N + jnp.arange(EP - E, dtype=jnp.int32) % (NPAD - N)
    src_p = jnp.concatenate([src, padi])
    dst_p = jnp.concatenate([dst, padi])

    x_p = jnp.zeros((NPAD, D), jnp.float32).at[:N].set(x)
    b1r = b1.reshape(1, D)
    b2r = b2.reshape(1, D)
    b3r = b3.reshape(1, D)

    degp = _agg_deg(dst_p)
    dinv, t1 = _tc_dinv_scale(degp[0], degp[1], x_p, W1)

    a1 = _agg_rows(t1, src_p, dst_p)
    t2 = _tc_combine(a1, t1, dinv, b1r, W2, relu_next=True)

    a2 = _agg_rows(t2, src_p, dst_p)
    t3 = _tc_combine(a2, t2, dinv, b2r, W3, relu_next=True)

    a3 = _agg_rows(t3, src_p, dst_p)
    out = _tc_combine(a3, t3, dinv, b3r, W3, relu_next=False)
    return out[:N]
